# Initial kernel scaffold; baseline (speedup 1.0000x reference)
import jax
import jax.numpy as jnp
from jax.experimental import pallas as pl


def kernel(x_nchw, weight, bias):
    raise NotImplementedError("write your pallas kernel here")


# speedup vs baseline: 1.1798x; 1.1798x over previous
import functools

import jax
import jax.numpy as jnp
from jax import lax
from jax.experimental import pallas as pl
from jax.experimental.pallas import tpu as pltpu


def _ct2d_kernel(xw_ref, w_ref, b_ref, o_ref, a_ref):
    C = xw_ref.shape[2]
    OW = o_ref.shape[3]
    bh = o_ref.shape[2] // 2
    a0 = pl.program_id(1) * bh
    bias = b_ref[...]
    w0 = w_ref[0]
    w1 = w_ref[1]

    def build_row(t, carry):
        row = xw_ref[0, a0 + t, :, :]
        for kw in range(4):
            a_ref[pl.ds((4 * t + kw) * C, C), :] = row[:, kw:kw + OW]
        return carry

    lax.fori_loop(0, bh + 2, build_row, 0, unroll=2)

    def row_body(l, carry):
        p0 = a_ref[pl.ds(l * 4 * C, 8 * C), :]
        p1 = a_ref[pl.ds((l + 1) * 4 * C, 8 * C), :]
        y0 = jnp.dot(w0, p0, preferred_element_type=jnp.float32) + bias
        y1 = jnp.dot(w1, p1, preferred_element_type=jnp.float32) + bias
        o_ref[0, :, 2 * l, :] = y0
        o_ref[0, :, 2 * l + 1, :] = y1
        return carry

    lax.fori_loop(0, bh, row_body, 0, unroll=2)


@functools.partial(jax.jit, static_argnames=("block_h",))
def _forward(x_nchw, weight, bias, *, block_h=16):
    N, C, H, W = x_nchw.shape
    OH, OW = 2 * H, 2 * W
    WD = 2 * W + 3

    bh = block_h
    while H % bh:
        bh //= 2
    n_hb = H // bh

    xt = jnp.transpose(x_nchw, (0, 2, 1, 3))
    x_il = jnp.stack([xt, jnp.zeros_like(xt)], axis=-1).reshape(N, H, C, 2 * W)
    xw = jnp.pad(x_il, ((0, 0), (1, 1), (0, 0), (2, 1))).astype(jnp.bfloat16)

    wp = []
    for dy in (0, 1):
        taps = []
        for di in (0, 1):
            kh = 3 - dy - 2 * di
            for kw in range(4):
                taps.append(weight[:, :, kh, 3 - kw])
        wp.append(jnp.stack(taps, axis=0).reshape(8 * C, C).T)
    w_all = jnp.stack(wp, axis=0).astype(jnp.bfloat16)
    b2d = bias.reshape(C, 1).astype(jnp.float32)

    return pl.pallas_call(
        _ct2d_kernel,
        out_shape=jax.ShapeDtypeStruct((N, C, OH, OW), x_nchw.dtype),
        grid=(N, n_hb),
        in_specs=[
            pl.BlockSpec((1, H + 2, C, WD), lambda n, h: (n, 0, 0, 0)),
            pl.BlockSpec((2, C, 8 * C), lambda n, h: (0, 0, 0)),
            pl.BlockSpec((C, 1), lambda n, h: (0, 0)),
        ],
        out_specs=pl.BlockSpec((1, C, 2 * bh, OW), lambda n, h: (n, 0, h, 0)),
        scratch_shapes=[pltpu.VMEM(((bh + 2) * 4 * C, OW), jnp.bfloat16)],
        compiler_params=pltpu.CompilerParams(
            dimension_semantics=("parallel", "parallel")),
    )(xw, w_all, b2d)


def kernel(x_nchw, weight, bias):
    return _forward(x_nchw, weight, bias)


# speedup vs baseline: 1.2895x; 1.0929x over previous
import functools

import jax
import jax.numpy as jnp
from jax import lax
from jax.experimental import pallas as pl
from jax.experimental.pallas import tpu as pltpu


def _ct2d_kernel(xw_ref, w_ref, b_ref, o_ref, a_ref):
    C = xw_ref.shape[2]
    OW = o_ref.shape[3]
    bh = o_ref.shape[2] // 2
    a0 = pl.program_id(1) * bh
    bias = b_ref[...]
    w_cat = w_ref[...]

    def build_row(t, carry):
        row = xw_ref[0, a0 + t, :, :]
        for kw in range(4):
            a_ref[pl.ds((4 * t + kw) * C, C), :] = row[:, kw:kw + OW]
        return carry

    lax.fori_loop(0, bh + 2, build_row, 0, unroll=2)

    def row_body(t, carry):
        q = a_ref[pl.ds(t * 4 * C, 8 * C), :]
        y = jnp.dot(w_cat, q, preferred_element_type=jnp.float32) + bias
        o_ref[0, :, 2 * t, :] = y[:C]
        o_ref[0, :, 2 * t - 1, :] = y[C:]
        return carry

    lax.fori_loop(1, bh, row_body, 0, unroll=4)

    q0 = a_ref[pl.ds(0, 8 * C), :]
    o_ref[0, :, 0, :] = (
        jnp.dot(w_cat[:C], q0, preferred_element_type=jnp.float32) + bias[:C])
    qb = a_ref[pl.ds(bh * 4 * C, 8 * C), :]
    o_ref[0, :, 2 * bh - 1, :] = (
        jnp.dot(w_cat[C:], qb, preferred_element_type=jnp.float32) + bias[C:])


@functools.partial(jax.jit, static_argnames=("block_h",))
def _forward(x_nchw, weight, bias, *, block_h=16):
    N, C, H, W = x_nchw.shape
    OH, OW = 2 * H, 2 * W
    WD = 2 * W + 3

    bh = block_h
    while H % bh:
        bh //= 2
    n_hb = H // bh

    xt = jnp.transpose(x_nchw, (0, 2, 1, 3))
    x_il = jnp.stack([xt, jnp.zeros_like(xt)], axis=-1).reshape(N, H, C, 2 * W)
    xw = jnp.pad(x_il, ((0, 0), (1, 1), (0, 0), (2, 1))).astype(jnp.bfloat16)

    wp = []
    for dy in (0, 1):
        taps = []
        for di in (0, 1):
            kh = 3 - dy - 2 * di
            for kw in range(4):
                taps.append(weight[:, :, kh, 3 - kw])
        wp.append(jnp.stack(taps, axis=0).reshape(8 * C, C).T)
    w_all = jnp.concatenate(wp, axis=0).astype(jnp.bfloat16)
    b2d = jnp.concatenate([bias, bias]).reshape(2 * C, 1).astype(jnp.float32)

    return pl.pallas_call(
        _ct2d_kernel,
        out_shape=jax.ShapeDtypeStruct((N, C, OH, OW), x_nchw.dtype),
        grid=(N, n_hb),
        in_specs=[
            pl.BlockSpec((1, H + 2, C, WD), lambda n, h: (n, 0, 0, 0)),
            pl.BlockSpec((2 * C, 8 * C), lambda n, h: (0, 0)),
            pl.BlockSpec((2 * C, 1), lambda n, h: (0, 0)),
        ],
        out_specs=pl.BlockSpec((1, C, 2 * bh, OW), lambda n, h: (n, 0, h, 0)),
        scratch_shapes=[pltpu.VMEM(((bh + 2) * 4 * C, OW), jnp.bfloat16)],
        compiler_params=pltpu.CompilerParams(
            dimension_semantics=("parallel", "parallel")),
    )(xw, w_all, b2d)


def kernel(x_nchw, weight, bias):
    return _forward(x_nchw, weight, bias)
